# baseline (speedup 1.0000x reference)
import functools
import math

import jax
import jax.numpy as jnp
from jax import lax
from jax.experimental import pallas as pl
from jax.experimental.pallas import tpu as pltpu
from jax.experimental.pallas import tpu_sc as plsc

D_MODEL = 1024
SCALE = math.sqrt(D_MODEL)
LANES = 16
CHUNK = 16
NBUF = 4


@functools.partial(jax.jit, static_argnums=(2, 3))
def _embed_lookup(x_flat, table, n_total, n_workers):
    n_per_w = n_total // n_workers
    n_chunks = n_per_w // CHUNK
    mesh = plsc.VectorSubcoreMesh(core_axis_name="c", subcore_axis_name="s")

    @functools.partial(
        pl.kernel,
        mesh=mesh,
        out_type=jax.ShapeDtypeStruct((n_total, D_MODEL), jnp.float32),
        scratch_types=[
            pltpu.VMEM((n_per_w,), jnp.int32),
            pltpu.VMEM((NBUF, CHUNK, D_MODEL), jnp.float32),
            pltpu.SemaphoreType.DMA((NBUF,)),
            pltpu.SemaphoreType.DMA((NBUF,)),
        ],
    )
    def k(x_hbm, table_hbm, out_hbm, idx_v, bufs, in_sem, out_sem):
        num_c = 2
        wid = lax.axis_index("s") * num_c + lax.axis_index("c")
        base = wid * n_per_w
        pltpu.sync_copy(x_hbm.at[pl.ds(base, n_per_w)], idx_v)

        def gather(c, b):
            pltpu.async_copy(
                table_hbm.at[idx_v.at[pl.ds(c * CHUNK, CHUNK)]],
                bufs.at[b],
                in_sem.at[b],
            )

        def scale_and_emit(c, b):
            pltpu.make_async_copy(
                table_hbm.at[idx_v.at[pl.ds(0, CHUNK)]], bufs.at[b], in_sem.at[b]
            ).wait()

            @plsc.parallel_loop(0, CHUNK, unroll=2)
            def row_body(i):
                for j in range(D_MODEL // LANES):
                    sl = pl.ds(j * LANES, LANES)
                    bufs[b, i, sl] = bufs[b, i, sl] * SCALE
            pltpu.async_copy(
                bufs.at[b], out_hbm.at[pl.ds(base + c * CHUNK, CHUNK)], out_sem.at[b]
            )

        def wait_out(b):
            pltpu.make_async_copy(
                bufs.at[b], out_hbm.at[pl.ds(base, CHUNK)], out_sem.at[b]
            ).wait()

        gather(0, 0)
        gather(1, 1)
        gather(2, 2)
        scale_and_emit(0, 0)
        gather(3, 3)
        scale_and_emit(1, 1)
        wait_out(0)
        gather(4, 0)
        scale_and_emit(2, 2)
        wait_out(1)
        gather(5, 1)
        scale_and_emit(3, 3)

        def group_body(i, carry):
            c0 = i * NBUF
            for b in range(NBUF):
                c = c0 + b
                wait_out((b + 2) % NBUF)
                gather(c + 2, (b + 2) % NBUF)
                scale_and_emit(c, b)
            return carry

        lax.fori_loop(1, n_chunks // NBUF - 1, group_body, 0)

        cl = n_chunks - NBUF
        wait_out(2)
        gather(cl + 2, 2)
        scale_and_emit(cl, 0)
        wait_out(3)
        gather(cl + 3, 3)
        scale_and_emit(cl + 1, 1)
        scale_and_emit(cl + 2, 2)
        scale_and_emit(cl + 3, 3)
        for b in range(NBUF):
            wait_out(b)

    return k(x_flat, table)


def kernel(x, table):
    b, s = x.shape
    n_total = b * s
    out = _embed_lookup(x.reshape(n_total).astype(jnp.int32), table, n_total, 32)
    return out.reshape(b, s, D_MODEL)


# speedup vs baseline: 1.1369x; 1.1369x over previous
import functools
import math

import jax
import jax.numpy as jnp
from jax import lax
from jax.experimental import pallas as pl
from jax.experimental.pallas import tpu as pltpu
from jax.experimental.pallas import tpu_sc as plsc

D_MODEL = 1024
SCALE = math.sqrt(D_MODEL)
LANES = 16
CHUNK = 16
NBUF = 4


@functools.partial(jax.jit, static_argnums=(2, 3))
def _embed_lookup(x_flat, table, n_total, n_workers):
    n_per_w = n_total // n_workers
    n_chunks = n_per_w // CHUNK
    mesh = plsc.VectorSubcoreMesh(core_axis_name="c", subcore_axis_name="s")

    @functools.partial(
        pl.kernel,
        mesh=mesh,
        out_type=jax.ShapeDtypeStruct((n_total, D_MODEL), jnp.float32),
        scratch_types=[
            pltpu.VMEM((n_per_w,), jnp.int32),
            pltpu.VMEM((NBUF, CHUNK, D_MODEL), jnp.float32),
            pltpu.SemaphoreType.DMA((NBUF,)),
            pltpu.SemaphoreType.DMA((NBUF,)),
        ],
    )
    def k(x_hbm, table_hbm, out_hbm, idx_v, bufs, in_sem, out_sem):
        num_c = 2
        wid = lax.axis_index("s") * num_c + lax.axis_index("c")
        base = wid * n_per_w
        pltpu.sync_copy(x_hbm.at[pl.ds(base, n_per_w)], idx_v)

        def gather(c, b):
            pltpu.async_copy(
                table_hbm.at[idx_v.at[pl.ds(c * CHUNK, CHUNK)]],
                bufs.at[b],
                in_sem.at[b],
            )

        def scale_and_emit(c, b):
            pltpu.make_async_copy(
                table_hbm.at[idx_v.at[pl.ds(0, CHUNK)]], bufs.at[b], in_sem.at[b]
            ).wait()

            @plsc.parallel_loop(0, CHUNK, unroll=1)
            def row_body(i):
                for j in range(D_MODEL // LANES):
                    sl = pl.ds(j * LANES, LANES)
                    bufs[b, i, sl] = bufs[b, i, sl] * SCALE
            pltpu.async_copy(
                bufs.at[b], out_hbm.at[pl.ds(base + c * CHUNK, CHUNK)], out_sem.at[b]
            )

        def wait_out(b):
            pltpu.make_async_copy(
                bufs.at[b], out_hbm.at[pl.ds(base, CHUNK)], out_sem.at[b]
            ).wait()

        gather(0, 0)
        gather(1, 1)
        gather(2, 2)
        scale_and_emit(0, 0)
        gather(3, 3)
        scale_and_emit(1, 1)
        wait_out(0)
        gather(4, 0)
        scale_and_emit(2, 2)
        wait_out(1)
        gather(5, 1)
        scale_and_emit(3, 3)

        def group_body(i, carry):
            c0 = i * NBUF
            for b in range(NBUF):
                c = c0 + b
                wait_out((b + 2) % NBUF)
                gather(c + 2, (b + 2) % NBUF)
                scale_and_emit(c, b)
            return carry

        lax.fori_loop(1, n_chunks // NBUF - 1, group_body, 0)

        cl = n_chunks - NBUF
        wait_out(2)
        gather(cl + 2, 2)
        scale_and_emit(cl, 0)
        wait_out(3)
        gather(cl + 3, 3)
        scale_and_emit(cl + 1, 1)
        scale_and_emit(cl + 2, 2)
        scale_and_emit(cl + 3, 3)
        for b in range(NBUF):
            wait_out(b)

    return k(x_flat, table)


def kernel(x, table):
    b, s = x.shape
    n_total = b * s
    out = _embed_lookup(x.reshape(n_total).astype(jnp.int32), table, n_total, 32)
    return out.reshape(b, s, D_MODEL)
